# Initial kernel scaffold; baseline (speedup 1.0000x reference)
import jax
import jax.numpy as jnp
from jax.experimental import pallas as pl


def kernel(features, pos, cam_ids):
    raise NotImplementedError("write your pallas kernel here")


# speedup vs baseline: 5.9278x; 5.9278x over previous
import functools

import jax
import jax.numpy as jnp
from jax import lax
from jax.experimental import pallas as pl
from jax.experimental.pallas import tpu as pltpu, tpu_sc as plsc

B, N, C = 8, 1024, 384
K = 8
M = N // 5
MPAD = 256
R = 256
NB = N // R
ROWS = B * M
ROWS_PAD = 2048
AUXW = 128


def _sq_body(x_ref, sq_ref):
    x = x_ref[0]
    x2 = x * x
    acc = (x2[:, 0:128] + x2[:, 128:256]) + x2[:, 256:384]
    g8 = acc[:, 0:8]
    for kk in range(1, 16):
        g8 = g8 + acc[:, 8 * kk: 8 * kk + 8]
    f4 = g8[:, 0:4] + g8[:, 4:8]
    e2 = f4[:, 0:2] + f4[:, 2:4]
    sq_ref[0, 0, :] = e2[:, 0] + e2[:, 1]


def _density_body(xb_ref, xf_ref, sq_ref, sqb_ref, dens_ref):
    xb = xb_ref[0]
    xf = xf_ref[0]
    sq = sq_ref[0][0]
    sqb = sqb_ref[0][0]

    g = lax.dot_general(xb, xf, (((1,), (1,)), ((), ())),
                        preferred_element_type=jnp.float32)
    d2 = sqb[:, None] + sq[None, :] - 2.0 * g

    iota = lax.broadcasted_iota(jnp.int32, (R, N), 1)
    vals = []
    for _ in range(K):
        m = jnp.min(d2, axis=1, keepdims=True)
        vals.append(m[:, 0])
        idx = jnp.min(jnp.where(d2 == m, iota, 2 * N), axis=1, keepdims=True)
        d2 = jnp.where(iota == idx, jnp.inf, d2)

    ds = [jnp.sqrt(jnp.maximum(v, 0.0)) for v in vals]
    s = ((ds[0] + ds[4]) + (ds[2] + ds[6])) + ((ds[1] + ds[5]) + (ds[3] + ds[7]))
    dens_ref[0, 0, :] = s / 8.0


def _rank_body(dens_ref, inds_ref):
    dens = dens_ref[0][0]
    dcol = dens[:, None]
    drow = dens[None, :]
    ii = lax.broadcasted_iota(jnp.int32, (N, N), 0)
    jj = lax.broadcasted_iota(jnp.int32, (N, N), 1)
    before = (drow > dcol) | ((drow == dcol) & (jj < ii))
    rank = jnp.sum(jnp.where(before, 1.0, 0.0), axis=1).astype(jnp.int32)

    mm = lax.broadcasted_iota(jnp.int32, (N, MPAD), 1)
    iv = lax.broadcasted_iota(jnp.int32, (N, MPAD), 0)
    contrib = jnp.where(rank[:, None] == mm, iv, 0)
    inds_ref[0, 0, :] = jnp.sum(contrib, axis=0)


def _tc_topk(features):
    sq = pl.pallas_call(
        _sq_body,
        grid=(B,),
        in_specs=[pl.BlockSpec((1, N, C), lambda b: (b, 0, 0))],
        out_specs=pl.BlockSpec((1, 1, N), lambda b: (b, 0, 0)),
        out_shape=jax.ShapeDtypeStruct((B, 1, N), jnp.float32),
    )(features)

    dens = pl.pallas_call(
        _density_body,
        grid=(B * NB,),
        in_specs=[
            pl.BlockSpec((1, R, C), lambda i: (i // NB, i % NB, 0)),
            pl.BlockSpec((1, N, C), lambda i: (i // NB, 0, 0)),
            pl.BlockSpec((1, 1, N), lambda i: (i // NB, 0, 0)),
            pl.BlockSpec((1, 1, R), lambda i: (i // NB, 0, i % NB)),
        ],
        out_specs=pl.BlockSpec((1, 1, R), lambda i: (i // NB, 0, i % NB)),
        out_shape=jax.ShapeDtypeStruct((B, 1, N), jnp.float32),
    )(features, features, sq, sq)

    inds = pl.pallas_call(
        _rank_body,
        grid=(B,),
        in_specs=[pl.BlockSpec((1, 1, N), lambda b: (b, 0, 0))],
        out_specs=pl.BlockSpec((1, 1, MPAD), lambda b: (b, 0, 0)),
        out_shape=jax.ShapeDtypeStruct((B, 1, MPAD), jnp.int32),
    )(dens)
    return inds[:, 0, :M]


def _sc_gather(feat_table, aux_table, idx_flat):
    info = plsc.get_sparse_core_info()
    nw = info.num_cores * info.num_subcores
    rows_per_w = ROWS_PAD // nw

    @functools.partial(
        pl.kernel,
        mesh=plsc.VectorSubcoreMesh(core_axis_name="c", subcore_axis_name="s"),
        out_type=[
            jax.ShapeDtypeStruct((ROWS_PAD, C), jnp.float32),
            jax.ShapeDtypeStruct((ROWS_PAD, AUXW), jnp.float32),
        ],
        scratch_types=[
            pltpu.VMEM((rows_per_w,), jnp.int32),
            pltpu.VMEM((rows_per_w, C), jnp.float32),
            pltpu.VMEM((rows_per_w, AUXW), jnp.float32),
            pltpu.SemaphoreType.DMA,
        ],
    )
    def gather_k(feat_hbm, aux_hbm, idx_hbm, out_f_hbm, out_a_hbm,
                 idx_v, feat_v, aux_v, sem):
        wid = lax.axis_index("s") * info.num_cores + lax.axis_index("c")
        base = wid * rows_per_w
        pltpu.sync_copy(idx_hbm.at[pl.ds(base, rows_per_w)], idx_v)
        pltpu.async_copy(feat_hbm.at[idx_v], feat_v, sem).wait()
        pltpu.sync_copy(feat_v, out_f_hbm.at[pl.ds(base, rows_per_w)])
        pltpu.async_copy(aux_hbm.at[idx_v], aux_v, sem).wait()
        pltpu.sync_copy(aux_v, out_a_hbm.at[pl.ds(base, rows_per_w)])

    return gather_k(feat_table, aux_table, idx_flat)


def kernel(features, pos, cam_ids):
    inds = _tc_topk(features)

    flat = (inds + (jnp.arange(B, dtype=jnp.int32) * N)[:, None]).reshape(-1)
    idx_flat = jnp.concatenate(
        [flat, jnp.zeros((ROWS_PAD - ROWS,), jnp.int32)])

    feat_table = features.reshape(B * N, C)
    camf = cam_ids.astype(jnp.float32).reshape(B * N, 1)
    aux_table = jnp.concatenate(
        [pos.reshape(B * N, 3), camf,
         jnp.zeros((B * N, AUXW - 4), jnp.float32)], axis=1)

    out_f, out_a = _sc_gather(feat_table, aux_table, idx_flat)

    sampled_features = out_f[:ROWS].reshape(B, M, C)
    sampled_pos = out_a[:ROWS, 0:3].reshape(B, M, 3)
    sampled_cam = out_a[:ROWS, 3].astype(jnp.int32).reshape(B, M)
    return (sampled_features, sampled_pos, sampled_cam)


# speedup vs baseline: 6.7632x; 1.1409x over previous
import functools

import jax
import jax.numpy as jnp
from jax import lax
from jax.experimental import pallas as pl
from jax.experimental.pallas import tpu as pltpu, tpu_sc as plsc

B, N, C = 8, 1024, 384
K = 8
M = N // 5
MPAD = 256
R = 256
NB = N // R
ROWS = B * M
ROWS_PAD = 2048
AUXW = 128


def _sq_body(x_ref, sq_ref):
    x = x_ref[0]
    x2 = x * x
    acc = (x2[:, 0:128] + x2[:, 128:256]) + x2[:, 256:384]
    g8 = acc[:, 0:8]
    for kk in range(1, 16):
        g8 = g8 + acc[:, 8 * kk: 8 * kk + 8]
    f4 = g8[:, 0:4] + g8[:, 4:8]
    e2 = f4[:, 0:2] + f4[:, 2:4]
    sq_ref[0, 0, :] = e2[:, 0] + e2[:, 1]


def _density_body(xb_ref, xf_ref, sq_ref, sqb_ref, dens_ref):
    xb = xb_ref[0]
    xf = xf_ref[0]
    sq = sq_ref[0][0]
    sqb = sqb_ref[0][0]

    g = lax.dot_general(xb, xf, (((1,), (1,)), ((), ())),
                        preferred_element_type=jnp.float32)
    d2 = sqb[:, None] + sq[None, :] - 2.0 * g

    ms, ps = [], []
    ptot = None
    for _ in range(K):
        m = jnp.min(d2, axis=1, keepdims=True)
        eq = d2 == m
        c = jnp.sum(jnp.where(eq, 1.0, 0.0), axis=1, keepdims=True)
        d2 = jnp.where(eq, jnp.inf, d2)
        ms.append(m[:, 0])
        ptot = c[:, 0] if ptot is None else ptot + c[:, 0]
        ps.append(ptot)

    vals = []
    for j in range(K):
        vj = ms[K - 1]
        for kk in range(K - 2, -1, -1):
            vj = jnp.where(ps[kk] > float(j), ms[kk], vj)
        vals.append(vj)

    ds = [jnp.sqrt(jnp.maximum(v, 0.0)) for v in vals]
    s = ((ds[0] + ds[4]) + (ds[2] + ds[6])) + ((ds[1] + ds[5]) + (ds[3] + ds[7]))
    dens_ref[0, 0, :] = s / 8.0


def _rank_body(dens_ref, inds_ref):
    dens = dens_ref[0][0]
    dcol = dens[:, None]
    drow = dens[None, :]
    ii = lax.broadcasted_iota(jnp.int32, (N, N), 0)
    jj = lax.broadcasted_iota(jnp.int32, (N, N), 1)
    before = (drow > dcol) | ((drow == dcol) & (jj < ii))
    rank = jnp.sum(jnp.where(before, 1.0, 0.0), axis=1).astype(jnp.int32)

    mm = lax.broadcasted_iota(jnp.int32, (N, MPAD), 1)
    iv = lax.broadcasted_iota(jnp.int32, (N, MPAD), 0)
    contrib = jnp.where(rank[:, None] == mm, iv, 0)
    inds_ref[0, 0, :] = jnp.sum(contrib, axis=0)


def _tc_topk(features):
    sq = pl.pallas_call(
        _sq_body,
        grid=(B,),
        in_specs=[pl.BlockSpec((1, N, C), lambda b: (b, 0, 0))],
        out_specs=pl.BlockSpec((1, 1, N), lambda b: (b, 0, 0)),
        out_shape=jax.ShapeDtypeStruct((B, 1, N), jnp.float32),
    )(features)

    dens = pl.pallas_call(
        _density_body,
        grid=(B * NB,),
        in_specs=[
            pl.BlockSpec((1, R, C), lambda i: (i // NB, i % NB, 0)),
            pl.BlockSpec((1, N, C), lambda i: (i // NB, 0, 0)),
            pl.BlockSpec((1, 1, N), lambda i: (i // NB, 0, 0)),
            pl.BlockSpec((1, 1, R), lambda i: (i // NB, 0, i % NB)),
        ],
        out_specs=pl.BlockSpec((1, 1, R), lambda i: (i // NB, 0, i % NB)),
        out_shape=jax.ShapeDtypeStruct((B, 1, N), jnp.float32),
    )(features, features, sq, sq)

    inds = pl.pallas_call(
        _rank_body,
        grid=(B,),
        in_specs=[pl.BlockSpec((1, 1, N), lambda b: (b, 0, 0))],
        out_specs=pl.BlockSpec((1, 1, MPAD), lambda b: (b, 0, 0)),
        out_shape=jax.ShapeDtypeStruct((B, 1, MPAD), jnp.int32),
    )(dens)
    return inds[:, 0, :M]


def _sc_gather(feat_table, aux_table, idx_flat):
    info = plsc.get_sparse_core_info()
    nw = info.num_cores * info.num_subcores
    rows_per_w = ROWS_PAD // nw

    @functools.partial(
        pl.kernel,
        mesh=plsc.VectorSubcoreMesh(core_axis_name="c", subcore_axis_name="s"),
        out_type=[
            jax.ShapeDtypeStruct((ROWS_PAD, C), jnp.float32),
            jax.ShapeDtypeStruct((ROWS_PAD, AUXW), jnp.float32),
        ],
        scratch_types=[
            pltpu.VMEM((rows_per_w,), jnp.int32),
            pltpu.VMEM((rows_per_w, C), jnp.float32),
            pltpu.VMEM((rows_per_w, AUXW), jnp.float32),
            pltpu.SemaphoreType.DMA,
            pltpu.SemaphoreType.DMA,
        ],
    )
    def gather_k(feat_hbm, aux_hbm, idx_hbm, out_f_hbm, out_a_hbm,
                 idx_v, feat_v, aux_v, sem_f, sem_a):
        wid = lax.axis_index("s") * info.num_cores + lax.axis_index("c")
        base = wid * rows_per_w
        pltpu.sync_copy(idx_hbm.at[pl.ds(base, rows_per_w)], idx_v)
        cp_f = pltpu.async_copy(feat_hbm.at[idx_v], feat_v, sem_f)
        cp_a = pltpu.async_copy(aux_hbm.at[idx_v], aux_v, sem_a)
        cp_f.wait()
        st_f = pltpu.async_copy(feat_v, out_f_hbm.at[pl.ds(base, rows_per_w)],
                                sem_f)
        cp_a.wait()
        st_a = pltpu.async_copy(aux_v, out_a_hbm.at[pl.ds(base, rows_per_w)],
                                sem_a)
        st_f.wait()
        st_a.wait()

    return gather_k(feat_table, aux_table, idx_flat)


def kernel(features, pos, cam_ids):
    inds = _tc_topk(features)

    flat = (inds + (jnp.arange(B, dtype=jnp.int32) * N)[:, None]).reshape(-1)
    idx_flat = jnp.concatenate(
        [flat, jnp.zeros((ROWS_PAD - ROWS,), jnp.int32)])

    feat_table = features.reshape(B * N, C)
    camf = cam_ids.astype(jnp.float32).reshape(B * N, 1)
    aux_table = jnp.concatenate(
        [pos.reshape(B * N, 3), camf,
         jnp.zeros((B * N, AUXW - 4), jnp.float32)], axis=1)

    out_f, out_a = _sc_gather(feat_table, aux_table, idx_flat)

    sampled_features = out_f[:ROWS].reshape(B, M, C)
    sampled_pos = out_a[:ROWS, 0:3].reshape(B, M, 3)
    sampled_cam = out_a[:ROWS, 3].astype(jnp.int32).reshape(B, M)
    return (sampled_features, sampled_pos, sampled_cam)


# speedup vs baseline: 7.2030x; 1.0650x over previous
import functools

import jax
import jax.numpy as jnp
from jax import lax
from jax.experimental import pallas as pl
from jax.experimental.pallas import tpu as pltpu, tpu_sc as plsc

B, N, C = 8, 1024, 384
K = 8
M = N // 5
MPAD = 256
R = 256
NB = N // R
ROWS = B * M
ROWS_PAD = 2048
AUXW = 128


def _sq_body(x_ref, sq_ref):
    x = x_ref[0]
    x2 = x * x
    acc = (x2[:, 0:128] + x2[:, 128:256]) + x2[:, 256:384]
    g8 = acc[:, 0:8]
    for kk in range(1, 16):
        g8 = g8 + acc[:, 8 * kk: 8 * kk + 8]
    f4 = g8[:, 0:4] + g8[:, 4:8]
    e2 = f4[:, 0:2] + f4[:, 2:4]
    sq_ref[0, 0, :] = e2[:, 0] + e2[:, 1]


def _density_body(xb_ref, xf_ref, sq_ref, sqb_ref, dens_ref):
    xb = xb_ref[0]
    xf = xf_ref[0]
    sq = sq_ref[0][0]
    sqb = sqb_ref[0][0]

    g = lax.dot_general(xb, xf, (((1,), (1,)), ((), ())),
                        preferred_element_type=jnp.float32)
    d2 = sqb[:, None] + sq[None, :] - 2.0 * g

    ms, ps = [], []
    ptot = None
    for _ in range(K):
        m = jnp.min(d2, axis=1, keepdims=True)
        eq = d2 == m
        c = jnp.sum(jnp.where(eq, 1.0, 0.0), axis=1, keepdims=True)
        d2 = jnp.where(eq, jnp.inf, d2)
        ms.append(m[:, 0])
        ptot = c[:, 0] if ptot is None else ptot + c[:, 0]
        ps.append(ptot)

    vals = []
    for j in range(K):
        vj = ms[K - 1]
        for kk in range(K - 2, -1, -1):
            vj = jnp.where(ps[kk] > float(j), ms[kk], vj)
        vals.append(vj)

    ds = [jnp.sqrt(jnp.maximum(v, 0.0)) for v in vals]
    s = ((ds[0] + ds[4]) + (ds[2] + ds[6])) + ((ds[1] + ds[5]) + (ds[3] + ds[7]))
    dens_ref[0, 0, :] = s / 8.0


def _rank_body(dens_ref, aux_ref, inds_ref, saux_ref):
    dens = dens_ref[0][0]
    aux = aux_ref[0]
    dcol = dens[:, None]
    drow = dens[None, :]
    ii = lax.broadcasted_iota(jnp.int32, (N, N), 0)
    jj = lax.broadcasted_iota(jnp.int32, (N, N), 1)
    before = (drow > dcol) | ((drow == dcol) & (jj < ii))
    rank = jnp.sum(jnp.where(before, 1.0, 0.0), axis=1).astype(jnp.int32)

    mm = lax.broadcasted_iota(jnp.int32, (N, MPAD), 1)
    iv = lax.broadcasted_iota(jnp.int32, (N, MPAD), 0)
    sel = rank[:, None] == mm
    inds_ref[0, 0, :] = jnp.sum(jnp.where(sel, iv, 0), axis=0)
    onehot = jnp.where(sel, 1.0, 0.0)
    saux_ref[0] = lax.dot_general(onehot, aux, (((0,), (0,)), ((), ())),
                                  preferred_element_type=jnp.float32)


def _tc_topk(features, aux):
    sq = pl.pallas_call(
        _sq_body,
        grid=(B,),
        in_specs=[pl.BlockSpec((1, N, C), lambda b: (b, 0, 0))],
        out_specs=pl.BlockSpec((1, 1, N), lambda b: (b, 0, 0)),
        out_shape=jax.ShapeDtypeStruct((B, 1, N), jnp.float32),
    )(features)

    dens = pl.pallas_call(
        _density_body,
        grid=(B * NB,),
        in_specs=[
            pl.BlockSpec((1, R, C), lambda i: (i // NB, i % NB, 0)),
            pl.BlockSpec((1, N, C), lambda i: (i // NB, 0, 0)),
            pl.BlockSpec((1, 1, N), lambda i: (i // NB, 0, 0)),
            pl.BlockSpec((1, 1, R), lambda i: (i // NB, 0, i % NB)),
        ],
        out_specs=pl.BlockSpec((1, 1, R), lambda i: (i // NB, 0, i % NB)),
        out_shape=jax.ShapeDtypeStruct((B, 1, N), jnp.float32),
    )(features, features, sq, sq)

    inds, saux = pl.pallas_call(
        _rank_body,
        grid=(B,),
        in_specs=[pl.BlockSpec((1, 1, N), lambda b: (b, 0, 0)),
                  pl.BlockSpec((1, N, 4), lambda b: (b, 0, 0))],
        out_specs=[pl.BlockSpec((1, 1, MPAD), lambda b: (b, 0, 0)),
                   pl.BlockSpec((1, MPAD, 4), lambda b: (b, 0, 0))],
        out_shape=[jax.ShapeDtypeStruct((B, 1, MPAD), jnp.int32),
                   jax.ShapeDtypeStruct((B, MPAD, 4), jnp.float32)],
    )(dens, aux)
    return inds[:, 0, :M], saux


def _sc_gather(feat_table, idx_flat):
    info = plsc.get_sparse_core_info()
    nw = info.num_cores * info.num_subcores
    rows_per_w = ROWS_PAD // nw

    @functools.partial(
        pl.kernel,
        mesh=plsc.VectorSubcoreMesh(core_axis_name="c", subcore_axis_name="s"),
        out_type=jax.ShapeDtypeStruct((ROWS_PAD, C), jnp.float32),
        scratch_types=[
            pltpu.VMEM((rows_per_w,), jnp.int32),
            pltpu.VMEM((rows_per_w, C), jnp.float32),
            pltpu.SemaphoreType.DMA,
        ],
    )
    def gather_k(feat_hbm, idx_hbm, out_f_hbm, idx_v, feat_v, sem_f):
        wid = lax.axis_index("s") * info.num_cores + lax.axis_index("c")
        base = wid * rows_per_w
        pltpu.sync_copy(idx_hbm.at[pl.ds(base, rows_per_w)], idx_v)
        pltpu.async_copy(feat_hbm.at[idx_v], feat_v, sem_f).wait()
        pltpu.sync_copy(feat_v, out_f_hbm.at[pl.ds(base, rows_per_w)])

    return gather_k(feat_table, idx_flat)


def kernel(features, pos, cam_ids):
    camf = cam_ids.astype(jnp.float32)[:, :, None]
    aux = jnp.concatenate([pos, camf], axis=2)

    inds, saux = _tc_topk(features, aux)

    flat = (inds + (jnp.arange(B, dtype=jnp.int32) * N)[:, None]).reshape(-1)
    idx_flat = jnp.concatenate(
        [flat, jnp.zeros((ROWS_PAD - ROWS,), jnp.int32)])

    out_f = _sc_gather(features.reshape(B * N, C), idx_flat)

    sampled_features = out_f[:ROWS].reshape(B, M, C)
    sampled_pos = saux[:, :M, 0:3]
    sampled_cam = saux[:, :M, 3].astype(jnp.int32)
    return (sampled_features, sampled_pos, sampled_cam)


# speedup vs baseline: 8.1079x; 1.1256x over previous
import functools

import jax
import jax.numpy as jnp
from jax import lax
from jax.experimental import pallas as pl
from jax.experimental.pallas import tpu as pltpu, tpu_sc as plsc

B, N, C = 8, 1024, 384
K = 8
M = N // 5
MPAD = 256
R = 256
NB = N // R
ROWS = B * M
ROWS_PAD = 2048
AUXW = 128


def _sq_body(x_ref, sq_ref):
    x = x_ref[0]
    x2 = x * x
    acc = (x2[:, 0:128] + x2[:, 128:256]) + x2[:, 256:384]
    accT = acc.T
    s8 = accT[0:8, :]
    for kk in range(1, 16):
        s8 = s8 + accT[8 * kk: 8 * kk + 8, :]
    f4 = s8[0:4, :] + s8[4:8, :]
    e2 = f4[0:2, :] + f4[2:4, :]
    sq_ref[0, 0, :] = e2[0, :] + e2[1, :]


def _density_body(xb_ref, xf_ref, sq_ref, sqb_ref, dens_ref):
    xb = xb_ref[0]
    xf = xf_ref[0]
    sq = sq_ref[0][0]
    sqb = sqb_ref[0][0]

    g = lax.dot_general(xb, xf, (((1,), (1,)), ((), ())),
                        preferred_element_type=jnp.float32)
    d2 = sqb[:, None] + sq[None, :] - 2.0 * g

    ms, ps = [], []
    ptot = None
    for _ in range(K):
        m = jnp.min(d2, axis=1, keepdims=True)
        eq = d2 == m
        c = jnp.sum(jnp.where(eq, 1.0, 0.0), axis=1, keepdims=True)
        d2 = jnp.where(eq, jnp.inf, d2)
        ms.append(m[:, 0])
        ptot = c[:, 0] if ptot is None else ptot + c[:, 0]
        ps.append(ptot)

    vals = []
    for j in range(K):
        vj = ms[K - 1]
        for kk in range(K - 2, -1, -1):
            vj = jnp.where(ps[kk] > float(j), ms[kk], vj)
        vals.append(vj)

    ds = [jnp.sqrt(jnp.maximum(v, 0.0)) for v in vals]
    s = ((ds[0] + ds[4]) + (ds[2] + ds[6])) + ((ds[1] + ds[5]) + (ds[3] + ds[7]))
    dens_ref[0, 0, :] = s / 8.0


def _rank_body(dens_ref, aux_ref, inds_ref, saux_ref):
    dens = dens_ref[0][0]
    aux = aux_ref[0]
    dcol = dens[:, None]
    drow = dens[None, :]
    ii = lax.broadcasted_iota(jnp.int32, (N, N), 0)
    jj = lax.broadcasted_iota(jnp.int32, (N, N), 1)
    before = (drow > dcol) | ((drow == dcol) & (jj < ii))
    rank = jnp.sum(jnp.where(before, 1.0, 0.0), axis=1).astype(jnp.int32)

    mm = lax.broadcasted_iota(jnp.int32, (N, MPAD), 1)
    iv = lax.broadcasted_iota(jnp.int32, (N, MPAD), 0)
    sel = rank[:, None] == mm
    inds_ref[0, 0, :] = jnp.sum(jnp.where(sel, iv, 0), axis=0)
    onehot = jnp.where(sel, 1.0, 0.0)
    saux_ref[0] = lax.dot_general(onehot, aux, (((0,), (0,)), ((), ())),
                                  preferred_element_type=jnp.float32)


def _tc_topk(features, aux):
    sq = pl.pallas_call(
        _sq_body,
        grid=(B,),
        in_specs=[pl.BlockSpec((1, N, C), lambda b: (b, 0, 0))],
        out_specs=pl.BlockSpec((1, 1, N), lambda b: (b, 0, 0)),
        out_shape=jax.ShapeDtypeStruct((B, 1, N), jnp.float32),
    )(features)

    dens = pl.pallas_call(
        _density_body,
        grid=(B * NB,),
        in_specs=[
            pl.BlockSpec((1, R, C), lambda i: (i // NB, i % NB, 0)),
            pl.BlockSpec((1, N, C), lambda i: (i // NB, 0, 0)),
            pl.BlockSpec((1, 1, N), lambda i: (i // NB, 0, 0)),
            pl.BlockSpec((1, 1, R), lambda i: (i // NB, 0, i % NB)),
        ],
        out_specs=pl.BlockSpec((1, 1, R), lambda i: (i // NB, 0, i % NB)),
        out_shape=jax.ShapeDtypeStruct((B, 1, N), jnp.float32),
    )(features, features, sq, sq)

    inds, saux = pl.pallas_call(
        _rank_body,
        grid=(B,),
        in_specs=[pl.BlockSpec((1, 1, N), lambda b: (b, 0, 0)),
                  pl.BlockSpec((1, N, 4), lambda b: (b, 0, 0))],
        out_specs=[pl.BlockSpec((1, 1, MPAD), lambda b: (b, 0, 0)),
                   pl.BlockSpec((1, MPAD, 4), lambda b: (b, 0, 0))],
        out_shape=[jax.ShapeDtypeStruct((B, 1, MPAD), jnp.int32),
                   jax.ShapeDtypeStruct((B, MPAD, 4), jnp.float32)],
    )(dens, aux)
    return inds[:, 0, :M], saux


def _sc_gather(feat_table, idx_flat):
    info = plsc.get_sparse_core_info()
    nw = info.num_cores * info.num_subcores
    rows_per_w = ROWS_PAD // nw

    @functools.partial(
        pl.kernel,
        mesh=plsc.VectorSubcoreMesh(core_axis_name="c", subcore_axis_name="s"),
        out_type=jax.ShapeDtypeStruct((ROWS_PAD, C), jnp.float32),
        scratch_types=[
            pltpu.VMEM((rows_per_w,), jnp.int32),
            pltpu.VMEM((rows_per_w // 2, C), jnp.float32),
            pltpu.VMEM((rows_per_w // 2, C), jnp.float32),
            pltpu.SemaphoreType.DMA,
            pltpu.SemaphoreType.DMA,
        ],
    )
    def gather_k(feat_hbm, idx_hbm, out_f_hbm, idx_v, f0_v, f1_v, sem0, sem1):
        wid = lax.axis_index("s") * info.num_cores + lax.axis_index("c")
        base = wid * rows_per_w
        half = rows_per_w // 2
        pltpu.sync_copy(idx_hbm.at[pl.ds(base, rows_per_w)], idx_v)
        g0 = pltpu.async_copy(feat_hbm.at[idx_v.at[pl.ds(0, half)]],
                              f0_v, sem0)
        g1 = pltpu.async_copy(feat_hbm.at[idx_v.at[pl.ds(half, half)]],
                              f1_v, sem1)
        g0.wait()
        s0 = pltpu.async_copy(f0_v, out_f_hbm.at[pl.ds(base, half)], sem0)
        g1.wait()
        s1 = pltpu.async_copy(f1_v, out_f_hbm.at[pl.ds(base + half, half)],
                              sem1)
        s0.wait()
        s1.wait()

    return gather_k(feat_table, idx_flat)


def kernel(features, pos, cam_ids):
    camf = cam_ids.astype(jnp.float32)[:, :, None]
    aux = jnp.concatenate([pos, camf], axis=2)

    inds, saux = _tc_topk(features, aux)

    flat = (inds + (jnp.arange(B, dtype=jnp.int32) * N)[:, None]).reshape(-1)
    idx_flat = jnp.concatenate(
        [flat, jnp.zeros((ROWS_PAD - ROWS,), jnp.int32)])

    out_f = _sc_gather(features.reshape(B * N, C), idx_flat)

    sampled_features = out_f[:ROWS].reshape(B, M, C)
    sampled_pos = saux[:, :M, 0:3]
    sampled_cam = saux[:, :M, 3].astype(jnp.int32)
    return (sampled_features, sampled_pos, sampled_cam)


# speedup vs baseline: 9.4103x; 1.1606x over previous
import functools

import jax
import jax.numpy as jnp
from jax import lax
from jax.experimental import pallas as pl
from jax.experimental.pallas import tpu as pltpu, tpu_sc as plsc

B, N, C = 8, 1024, 384
K = 8
M = N // 5
MPAD = 256
R = 256
NB = N // R
ROWS = B * M
ROWS_PAD = 2048
AUXW = 128


def _sq_body(x_ref, sq_ref):
    x = x_ref[0]
    x2 = x * x
    acc = (x2[:, 0:128] + x2[:, 128:256]) + x2[:, 256:384]
    accT = acc.T
    s8 = accT[0:8, :]
    for kk in range(1, 16):
        s8 = s8 + accT[8 * kk: 8 * kk + 8, :]
    f4 = s8[0:4, :] + s8[4:8, :]
    e2 = f4[0:2, :] + f4[2:4, :]
    sq_ref[0, 0, :] = e2[0, :] + e2[1, :]


def _density_body(xb_ref, xf_ref, sq_ref, sqb_ref, dens_ref):
    xb = xb_ref[0]
    xf = xf_ref[0]
    sq = sq_ref[0][0]
    sqb = sqb_ref[0][0]

    g = lax.dot_general(xb, xf, (((1,), (1,)), ((), ())),
                        preferred_element_type=jnp.float32)
    d2 = sqb[:, None] + sq[None, :] - 2.0 * g

    mcols, pcols = [], []
    ptot = None
    for _ in range(K):
        m = jnp.min(d2, axis=1, keepdims=True)
        eq = d2 == m
        c = jnp.sum(jnp.where(eq, 1.0, 0.0), axis=1, keepdims=True)
        d2 = jnp.where(eq, jnp.inf, d2)
        mcols.append(m)
        ptot = c if ptot is None else ptot + c
        pcols.append(ptot)
    t = jnp.concatenate(mcols + pcols, axis=1).T
    ms = [t[kk, :] for kk in range(K)]
    ps = [t[K + kk, :] for kk in range(K)]

    vals = []
    for j in range(K):
        vj = ms[K - 1]
        for kk in range(K - 2, -1, -1):
            vj = jnp.where(ps[kk] > float(j), ms[kk], vj)
        vals.append(vj)

    ds = [jnp.sqrt(jnp.maximum(v, 0.0)) for v in vals]
    s = ((ds[0] + ds[4]) + (ds[2] + ds[6])) + ((ds[1] + ds[5]) + (ds[3] + ds[7]))
    dens_ref[0, 0, :] = s / 8.0


def _rank_body(dens_ref, aux_ref, inds_ref, saux_ref):
    dens = dens_ref[0][0]
    aux = aux_ref[0]
    dcol = dens[:, None]
    drow = dens[None, :]
    ii = lax.broadcasted_iota(jnp.int32, (N, N), 0)
    jj = lax.broadcasted_iota(jnp.int32, (N, N), 1)
    before = (drow > dcol) | ((drow == dcol) & (jj < ii))
    rank = jnp.sum(jnp.where(before, 1.0, 0.0), axis=1).astype(jnp.int32)

    mm = lax.broadcasted_iota(jnp.int32, (N, MPAD), 1)
    iv = lax.broadcasted_iota(jnp.int32, (N, MPAD), 0)
    sel = rank[:, None] == mm
    inds_ref[0, 0, :] = jnp.sum(jnp.where(sel, iv, 0), axis=0)
    onehot = jnp.where(sel, 1.0, 0.0)
    saux_ref[0] = lax.dot_general(onehot, aux, (((0,), (0,)), ((), ())),
                                  preferred_element_type=jnp.float32)


def _tc_topk(features, aux):
    sq = pl.pallas_call(
        _sq_body,
        grid=(B,),
        in_specs=[pl.BlockSpec((1, N, C), lambda b: (b, 0, 0))],
        out_specs=pl.BlockSpec((1, 1, N), lambda b: (b, 0, 0)),
        out_shape=jax.ShapeDtypeStruct((B, 1, N), jnp.float32),
    )(features)

    dens = pl.pallas_call(
        _density_body,
        grid=(B * NB,),
        in_specs=[
            pl.BlockSpec((1, R, C), lambda i: (i // NB, i % NB, 0)),
            pl.BlockSpec((1, N, C), lambda i: (i // NB, 0, 0)),
            pl.BlockSpec((1, 1, N), lambda i: (i // NB, 0, 0)),
            pl.BlockSpec((1, 1, R), lambda i: (i // NB, 0, i % NB)),
        ],
        out_specs=pl.BlockSpec((1, 1, R), lambda i: (i // NB, 0, i % NB)),
        out_shape=jax.ShapeDtypeStruct((B, 1, N), jnp.float32),
    )(features, features, sq, sq)

    inds, saux = pl.pallas_call(
        _rank_body,
        grid=(B,),
        in_specs=[pl.BlockSpec((1, 1, N), lambda b: (b, 0, 0)),
                  pl.BlockSpec((1, N, 4), lambda b: (b, 0, 0))],
        out_specs=[pl.BlockSpec((1, 1, MPAD), lambda b: (b, 0, 0)),
                   pl.BlockSpec((1, MPAD, 4), lambda b: (b, 0, 0))],
        out_shape=[jax.ShapeDtypeStruct((B, 1, MPAD), jnp.int32),
                   jax.ShapeDtypeStruct((B, MPAD, 4), jnp.float32)],
    )(dens, aux)
    return inds[:, 0, :M], saux


def _sc_gather(feat_table, idx_flat):
    info = plsc.get_sparse_core_info()
    nw = info.num_cores * info.num_subcores
    rows_per_w = ROWS_PAD // nw

    @functools.partial(
        pl.kernel,
        mesh=plsc.VectorSubcoreMesh(core_axis_name="c", subcore_axis_name="s"),
        out_type=jax.ShapeDtypeStruct((ROWS_PAD, C), jnp.float32),
        scratch_types=[
            pltpu.VMEM((rows_per_w,), jnp.int32),
            pltpu.VMEM((rows_per_w // 2, C), jnp.float32),
            pltpu.VMEM((rows_per_w // 2, C), jnp.float32),
            pltpu.SemaphoreType.DMA,
            pltpu.SemaphoreType.DMA,
        ],
    )
    def gather_k(feat_hbm, idx_hbm, out_f_hbm, idx_v, f0_v, f1_v, sem0, sem1):
        wid = lax.axis_index("s") * info.num_cores + lax.axis_index("c")
        base = wid * rows_per_w
        half = rows_per_w // 2
        pltpu.sync_copy(idx_hbm.at[pl.ds(base, rows_per_w)], idx_v)
        g0 = pltpu.async_copy(feat_hbm.at[idx_v.at[pl.ds(0, half)]],
                              f0_v, sem0)
        g1 = pltpu.async_copy(feat_hbm.at[idx_v.at[pl.ds(half, half)]],
                              f1_v, sem1)
        g0.wait()
        s0 = pltpu.async_copy(f0_v, out_f_hbm.at[pl.ds(base, half)], sem0)
        g1.wait()
        s1 = pltpu.async_copy(f1_v, out_f_hbm.at[pl.ds(base + half, half)],
                              sem1)
        s0.wait()
        s1.wait()

    return gather_k(feat_table, idx_flat)


def kernel(features, pos, cam_ids):
    camf = cam_ids.astype(jnp.float32)[:, :, None]
    aux = jnp.concatenate([pos, camf], axis=2)

    inds, saux = _tc_topk(features, aux)

    flat = (inds + (jnp.arange(B, dtype=jnp.int32) * N)[:, None]).reshape(-1)
    idx_flat = jnp.concatenate(
        [flat, jnp.zeros((ROWS_PAD - ROWS,), jnp.int32)])

    out_f = _sc_gather(features.reshape(B * N, C), idx_flat)

    sampled_features = out_f[:ROWS].reshape(B, M, C)
    sampled_pos = saux[:, :M, 0:3]
    sampled_cam = saux[:, :M, 3].astype(jnp.int32)
    return (sampled_features, sampled_pos, sampled_cam)


# speedup vs baseline: 10.1278x; 1.0762x over previous
import functools

import jax
import jax.numpy as jnp
from jax import lax
from jax.experimental import pallas as pl
from jax.experimental.pallas import tpu as pltpu, tpu_sc as plsc

B, N, C = 8, 1024, 384
K = 8
M = N // 5
MPAD = 256
R = 256
NB = N // R
ROWS = B * M
ROWS_PAD = 2048
AUXW = 128


def _sq_body(x_ref, sq_ref):
    x = x_ref[0]
    x2 = x * x
    acc = (x2[:, 0:128] + x2[:, 128:256]) + x2[:, 256:384]
    accT = acc.T
    s8 = accT[0:8, :]
    for kk in range(1, 16):
        s8 = s8 + accT[8 * kk: 8 * kk + 8, :]
    f4 = s8[0:4, :] + s8[4:8, :]
    e2 = f4[0:2, :] + f4[2:4, :]
    sq_ref[0, 0, :] = e2[0, :] + e2[1, :]


def _select_body(xf_ref, sq_ref, aux_ref, inds_ref, saux_ref):
    xf = xf_ref[0]
    sq = sq_ref[0][0]
    aux = aux_ref[0]

    g = lax.dot_general(xf, xf, (((1,), (1,)), ((), ())),
                        preferred_element_type=jnp.float32)
    d2 = sq[:, None] + sq[None, :] - 2.0 * g

    mcols, pcols = [], []
    ptot = None
    for _ in range(K):
        m = jnp.min(d2, axis=1, keepdims=True)
        eq = d2 == m
        c = jnp.sum(jnp.where(eq, 1.0, 0.0), axis=1, keepdims=True)
        d2 = jnp.where(eq, jnp.inf, d2)
        mcols.append(m)
        ptot = c if ptot is None else ptot + c
        pcols.append(ptot)
    t = jnp.concatenate(mcols + pcols, axis=1).T
    ms = [t[kk, :] for kk in range(K)]
    ps = [t[K + kk, :] for kk in range(K)]

    vals = []
    for j in range(K):
        vj = ms[K - 1]
        for kk in range(K - 2, -1, -1):
            vj = jnp.where(ps[kk] > float(j), ms[kk], vj)
        vals.append(vj)

    ds = [jnp.sqrt(jnp.maximum(v, 0.0)) for v in vals]
    s = ((ds[0] + ds[4]) + (ds[2] + ds[6])) + ((ds[1] + ds[5]) + (ds[3] + ds[7]))
    dens = s / 8.0

    dcol = dens[:, None]
    drow = dens[None, :]
    ii = lax.broadcasted_iota(jnp.int32, (N, N), 0)
    jj = lax.broadcasted_iota(jnp.int32, (N, N), 1)
    before = (drow > dcol) | ((drow == dcol) & (jj < ii))
    rank = jnp.sum(jnp.where(before, 1.0, 0.0), axis=1).astype(jnp.int32)

    mm = lax.broadcasted_iota(jnp.int32, (N, MPAD), 1)
    iv = lax.broadcasted_iota(jnp.int32, (N, MPAD), 0)
    sel = rank[:, None] == mm
    inds_ref[0, 0, :] = jnp.sum(jnp.where(sel, iv, 0), axis=0)
    onehot = jnp.where(sel, 1.0, 0.0)
    saux_ref[0] = lax.dot_general(onehot, aux, (((0,), (0,)), ((), ())),
                                  preferred_element_type=jnp.float32)


def _tc_topk(features, aux):
    sq = pl.pallas_call(
        _sq_body,
        grid=(B,),
        in_specs=[pl.BlockSpec((1, N, C), lambda b: (b, 0, 0))],
        out_specs=pl.BlockSpec((1, 1, N), lambda b: (b, 0, 0)),
        out_shape=jax.ShapeDtypeStruct((B, 1, N), jnp.float32),
    )(features)

    inds, saux = pl.pallas_call(
        _select_body,
        grid=(B,),
        in_specs=[pl.BlockSpec((1, N, C), lambda b: (b, 0, 0)),
                  pl.BlockSpec((1, 1, N), lambda b: (b, 0, 0)),
                  pl.BlockSpec((1, N, 4), lambda b: (b, 0, 0))],
        out_specs=[pl.BlockSpec((1, 1, MPAD), lambda b: (b, 0, 0)),
                   pl.BlockSpec((1, MPAD, 4), lambda b: (b, 0, 0))],
        out_shape=[jax.ShapeDtypeStruct((B, 1, MPAD), jnp.int32),
                   jax.ShapeDtypeStruct((B, MPAD, 4), jnp.float32)],
    )(features, sq, aux)
    return inds[:, 0, :M], saux


def _sc_gather(feat_table, idx_flat):
    info = plsc.get_sparse_core_info()
    nw = info.num_cores * info.num_subcores
    rows_per_w = ROWS_PAD // nw

    @functools.partial(
        pl.kernel,
        mesh=plsc.VectorSubcoreMesh(core_axis_name="c", subcore_axis_name="s"),
        out_type=jax.ShapeDtypeStruct((ROWS_PAD, C), jnp.float32),
        scratch_types=[
            pltpu.VMEM((rows_per_w,), jnp.int32),
            pltpu.VMEM((rows_per_w // 2, C), jnp.float32),
            pltpu.VMEM((rows_per_w // 2, C), jnp.float32),
            pltpu.SemaphoreType.DMA,
            pltpu.SemaphoreType.DMA,
        ],
    )
    def gather_k(feat_hbm, idx_hbm, out_f_hbm, idx_v, f0_v, f1_v, sem0, sem1):
        wid = lax.axis_index("s") * info.num_cores + lax.axis_index("c")
        base = wid * rows_per_w
        half = rows_per_w // 2
        pltpu.sync_copy(idx_hbm.at[pl.ds(base, rows_per_w)], idx_v)
        g0 = pltpu.async_copy(feat_hbm.at[idx_v.at[pl.ds(0, half)]],
                              f0_v, sem0)
        g1 = pltpu.async_copy(feat_hbm.at[idx_v.at[pl.ds(half, half)]],
                              f1_v, sem1)
        g0.wait()
        s0 = pltpu.async_copy(f0_v, out_f_hbm.at[pl.ds(base, half)], sem0)
        g1.wait()
        s1 = pltpu.async_copy(f1_v, out_f_hbm.at[pl.ds(base + half, half)],
                              sem1)
        s0.wait()
        s1.wait()

    return gather_k(feat_table, idx_flat)


def kernel(features, pos, cam_ids):
    camf = cam_ids.astype(jnp.float32)[:, :, None]
    aux = jnp.concatenate([pos, camf], axis=2)

    inds, saux = _tc_topk(features, aux)

    flat = (inds + (jnp.arange(B, dtype=jnp.int32) * N)[:, None]).reshape(-1)
    idx_flat = jnp.concatenate(
        [flat, jnp.zeros((ROWS_PAD - ROWS,), jnp.int32)])

    out_f = _sc_gather(features.reshape(B * N, C), idx_flat)

    sampled_features = out_f[:ROWS].reshape(B, M, C)
    sampled_pos = saux[:, :M, 0:3]
    sampled_cam = saux[:, :M, 3].astype(jnp.int32)
    return (sampled_features, sampled_pos, sampled_cam)


# speedup vs baseline: 10.6993x; 1.0564x over previous
import functools

import jax
import jax.numpy as jnp
from jax import lax
from jax.experimental import pallas as pl
from jax.experimental.pallas import tpu as pltpu, tpu_sc as plsc

B, N, C = 8, 1024, 384
K = 8
M = N // 5
MPAD = 256
R = 256
NB = N // R
ROWS = B * M
ROWS_PAD = 2048
AUXW = 128


def _select_body(xf_ref, aux_ref, inds_ref, saux_ref):
    xf = xf_ref[0]
    aux = aux_ref[0]

    x2 = xf * xf
    acc = (x2[:, 0:128] + x2[:, 128:256]) + x2[:, 256:384]
    accT = acc.T
    s8 = accT[0:8, :]
    for kk in range(1, 16):
        s8 = s8 + accT[8 * kk: 8 * kk + 8, :]
    f4s = s8[0:4, :] + s8[4:8, :]
    e2s = f4s[0:2, :] + f4s[2:4, :]
    sq = e2s[0, :] + e2s[1, :]

    g = lax.dot_general(xf, xf, (((1,), (1,)), ((), ())),
                        preferred_element_type=jnp.float32)
    d2 = sq[:, None] + sq[None, :] - 2.0 * g

    mcols, pcols = [], []
    ptot = None
    for _ in range(K):
        m = jnp.min(d2, axis=1, keepdims=True)
        eq = d2 == m
        c = jnp.sum(jnp.where(eq, 1.0, 0.0), axis=1, keepdims=True)
        d2 = jnp.where(eq, jnp.inf, d2)
        mcols.append(m)
        ptot = c if ptot is None else ptot + c
        pcols.append(ptot)
    t = jnp.concatenate(mcols + pcols, axis=1).T
    ms = [t[kk, :] for kk in range(K)]
    ps = [t[K + kk, :] for kk in range(K)]

    vals = []
    for j in range(K):
        vj = ms[K - 1]
        for kk in range(K - 2, -1, -1):
            vj = jnp.where(ps[kk] > float(j), ms[kk], vj)
        vals.append(vj)

    ds = [jnp.sqrt(jnp.maximum(v, 0.0)) for v in vals]
    s = ((ds[0] + ds[4]) + (ds[2] + ds[6])) + ((ds[1] + ds[5]) + (ds[3] + ds[7]))
    dens = s / 8.0

    dcol = dens[:, None]
    drow = dens[None, :]
    ii = lax.broadcasted_iota(jnp.int32, (N, N), 0)
    jj = lax.broadcasted_iota(jnp.int32, (N, N), 1)
    before = (drow > dcol) | ((drow == dcol) & (jj < ii))
    rank = jnp.sum(jnp.where(before, 1.0, 0.0), axis=1).astype(jnp.int32)

    mm = lax.broadcasted_iota(jnp.int32, (N, MPAD), 1)
    iv = lax.broadcasted_iota(jnp.int32, (N, MPAD), 0)
    sel = rank[:, None] == mm
    inds_ref[0, 0, :] = jnp.sum(jnp.where(sel, iv, 0), axis=0)
    onehot = jnp.where(sel, 1.0, 0.0)
    saux_ref[0] = lax.dot_general(onehot, aux, (((0,), (0,)), ((), ())),
                                  preferred_element_type=jnp.float32)


def _tc_topk(features, aux):
    inds, saux = pl.pallas_call(
        _select_body,
        grid=(B,),
        in_specs=[pl.BlockSpec((1, N, C), lambda b: (b, 0, 0)),
                  pl.BlockSpec((1, N, 4), lambda b: (b, 0, 0))],
        out_specs=[pl.BlockSpec((1, 1, MPAD), lambda b: (b, 0, 0)),
                   pl.BlockSpec((1, MPAD, 4), lambda b: (b, 0, 0))],
        out_shape=[jax.ShapeDtypeStruct((B, 1, MPAD), jnp.int32),
                   jax.ShapeDtypeStruct((B, MPAD, 4), jnp.float32)],
    )(features, aux)
    return inds[:, 0, :M], saux


def _sc_gather(feat_table, idx_flat):
    info = plsc.get_sparse_core_info()
    nw = info.num_cores * info.num_subcores
    rows_per_w = ROWS_PAD // nw

    @functools.partial(
        pl.kernel,
        mesh=plsc.VectorSubcoreMesh(core_axis_name="c", subcore_axis_name="s"),
        out_type=jax.ShapeDtypeStruct((ROWS_PAD, C), jnp.float32),
        scratch_types=[
            pltpu.VMEM((rows_per_w,), jnp.int32),
            pltpu.VMEM((rows_per_w // 2, C), jnp.float32),
            pltpu.VMEM((rows_per_w // 2, C), jnp.float32),
            pltpu.SemaphoreType.DMA,
            pltpu.SemaphoreType.DMA,
        ],
    )
    def gather_k(feat_hbm, idx_hbm, out_f_hbm, idx_v, f0_v, f1_v, sem0, sem1):
        wid = lax.axis_index("s") * info.num_cores + lax.axis_index("c")
        base = wid * rows_per_w
        half = rows_per_w // 2
        pltpu.sync_copy(idx_hbm.at[pl.ds(base, rows_per_w)], idx_v)
        g0 = pltpu.async_copy(feat_hbm.at[idx_v.at[pl.ds(0, half)]],
                              f0_v, sem0)
        g1 = pltpu.async_copy(feat_hbm.at[idx_v.at[pl.ds(half, half)]],
                              f1_v, sem1)
        g0.wait()
        s0 = pltpu.async_copy(f0_v, out_f_hbm.at[pl.ds(base, half)], sem0)
        g1.wait()
        s1 = pltpu.async_copy(f1_v, out_f_hbm.at[pl.ds(base + half, half)],
                              sem1)
        s0.wait()
        s1.wait()

    return gather_k(feat_table, idx_flat)


def kernel(features, pos, cam_ids):
    camf = cam_ids.astype(jnp.float32)[:, :, None]
    aux = jnp.concatenate([pos, camf], axis=2)

    inds, saux = _tc_topk(features, aux)

    flat = (inds + (jnp.arange(B, dtype=jnp.int32) * N)[:, None]).reshape(-1)
    idx_flat = jnp.concatenate(
        [flat, jnp.zeros((ROWS_PAD - ROWS,), jnp.int32)])

    out_f = _sc_gather(features.reshape(B * N, C), idx_flat)

    sampled_features = out_f[:ROWS].reshape(B, M, C)
    sampled_pos = saux[:, :M, 0:3]
    sampled_cam = saux[:, :M, 3].astype(jnp.int32)
    return (sampled_features, sampled_pos, sampled_cam)
